# Initial kernel scaffold; baseline (speedup 1.0000x reference)
import functools

import jax
import jax.numpy as jnp
from jax import lax
from jax.experimental import pallas as pl
from jax.experimental.pallas import tpu as pltpu
from jax.experimental.pallas import tpu_sc as plsc

_N = 10000
_B = 256
_D = 256
_HALF = 128
_NACC = 10016
_RPT = 626
_R = 400
_G = _N // _R

_f32 = jnp.float32


def _sc_mesh():
    return plsc.VectorSubcoreMesh(core_axis_name="c", subcore_axis_name="s")


def _seg_sum_sc(tab_l, tab_r, src2d, dst2d, zeros_l):
    ep = src2d.shape[0]
    cpt = ep // 16

    @functools.partial(
        pl.kernel,
        out_type=[jax.ShapeDtypeStruct((_N, _HALF), _f32)] * 2,
        mesh=_sc_mesh(),
        scratch_types=[
            pltpu.VMEM((128,), jnp.int32),
            pltpu.VMEM((128,), jnp.int32),
            pltpu.VMEM((128, _HALF), _f32),
            pltpu.VMEM_SHARED((_NACC, _HALF), _f32),
            pltpu.SemaphoreType.DMA,
        ],
    )
    def k(tabl_h, tabr_h, src_h, dst_h, z_h, outl_h, outr_h,
          idxg, idxs, rows, acc, sem):
        c = lax.axis_index("c")
        s = lax.axis_index("s")
        r0 = s * _RPT
        pltpu.sync_copy(z_h.at[pl.ds(r0, _RPT)], acc.at[pl.ds(r0, _RPT)])
        plsc.subcore_barrier()

        def run(tab_h):
            def body(i, carry):
                row = s * cpt + i
                pltpu.sync_copy(src_h.at[row], idxg)
                pltpu.async_copy(tab_h.at[idxg], rows, sem).wait()
                pltpu.sync_copy(dst_h.at[row], idxs)
                pltpu.sync_copy(rows, acc.at[idxs], add=True)
                return carry
            lax.fori_loop(0, cpt, body, 0)

        @pl.when(c == 0)
        def _():
            run(tabl_h)

        @pl.when(c == 1)
        def _():
            run(tabr_h)

        plsc.subcore_barrier()
        rd = jnp.minimum(r0, _N - _RPT)

        @pl.when(c == 0)
        def _():
            pltpu.sync_copy(acc.at[pl.ds(rd, _RPT)], outl_h.at[pl.ds(rd, _RPT)])

        @pl.when(c == 1)
        def _():
            pltpu.sync_copy(acc.at[pl.ds(rd, _RPT)], outr_h.at[pl.ds(rd, _RPT)])

    return k(tab_l, tab_r, src2d, dst2d, zeros_l)


def _counts_sc(ea_pad, dst2d, ones_src, col2d, zeros16):
    epa = dst2d.shape[0]
    epb = col2d.shape[0]
    cpa = epa // 32
    cpb = epb // 32

    @functools.partial(
        pl.kernel,
        out_type=[jax.ShapeDtypeStruct((_N, 16), _f32)] * 4,
        mesh=_sc_mesh(),
        scratch_types=[
            pltpu.VMEM((128,), jnp.int32),
            pltpu.VMEM((128, 16), _f32),
            pltpu.VMEM_SHARED((_NACC, 16), _f32),
            pltpu.VMEM_SHARED((_NACC, 16), _f32),
            pltpu.SemaphoreType.DMA,
        ],
    )
    def k(ea_h, dst_h, ones_h, col_h, z_h, a0_h, a1_h, c0_h, c1_h,
          idxs, rows, acc_a, acc_b, sem):
        c = lax.axis_index("c")
        s = lax.axis_index("s")
        r0 = s * _RPT
        pltpu.sync_copy(z_h.at[pl.ds(r0, _RPT)], acc_a.at[pl.ds(r0, _RPT)])
        pltpu.sync_copy(z_h.at[pl.ds(r0, _RPT)], acc_b.at[pl.ds(r0, _RPT)])
        plsc.subcore_barrier()
        w = c * 16 + s

        def body_a(i, carry):
            row = w * cpa + i
            pltpu.sync_copy(ea_h.at[pl.ds(row * 128, 128)], rows)
            pltpu.sync_copy(dst_h.at[row], idxs)
            pltpu.sync_copy(rows, acc_a.at[idxs], add=True)
            return carry
        lax.fori_loop(0, cpa, body_a, 0)

        pltpu.sync_copy(ones_h, rows)

        def body_b(i, carry):
            row = w * cpb + i
            pltpu.sync_copy(col_h.at[row], idxs)
            pltpu.sync_copy(rows, acc_b.at[idxs], add=True)
            return carry
        lax.fori_loop(0, cpb, body_b, 0)

        plsc.subcore_barrier()
        rd = jnp.minimum(r0, _N - _RPT)

        @pl.when(c == 0)
        def _():
            pltpu.sync_copy(acc_a.at[pl.ds(rd, _RPT)], a0_h.at[pl.ds(rd, _RPT)])
            pltpu.sync_copy(acc_b.at[pl.ds(rd, _RPT)], c0_h.at[pl.ds(rd, _RPT)])

        @pl.when(c == 1)
        def _():
            pltpu.sync_copy(acc_a.at[pl.ds(rd, _RPT)], a1_h.at[pl.ds(rd, _RPT)])
            pltpu.sync_copy(acc_b.at[pl.ds(rd, _RPT)], c1_h.at[pl.ds(rd, _RPT)])

    return k(ea_pad, dst2d, ones_src, col2d, zeros16)


def _row_spec(width):
    return pl.BlockSpec((_R, width), lambda i: (i, 0))


def _full_spec(shape):
    nd = len(shape)
    return pl.BlockSpec(shape, lambda i: (0,) * nd)


def _tc_emb(x, w, b):
    d_in = x.shape[1]

    def body(x_r, w_r, b_r, hl_r, hr_r):
        z = jnp.dot(x_r[...], w_r[...], preferred_element_type=_f32) + b_r[...]
        z = jnp.maximum(z, 0.0)
        hl_r[...] = z[:, :_HALF]
        hr_r[...] = z[:, _HALF:]

    return pl.pallas_call(
        body,
        grid=(_G,),
        in_specs=[_row_spec(d_in), _full_spec((d_in, _D)), _full_spec((1, _D))],
        out_specs=[_row_spec(_HALF)] * 2,
        out_shape=[jax.ShapeDtypeStruct((_N, _HALF), _f32)] * 2,
    )(x, w, b.reshape(1, -1))


def _tc_layer(hl, hr, aggl, aggr, a0, a1, wp, w1, b1, w2, b2):
    def body(hl_r, hr_r, al_r, ar_r, a0_r, a1_r, wp_r, w1_r, b1_r, w2_r, b2_r,
             z2_r, st_r):
        i = pl.program_id(0)
        h = jnp.concatenate([hl_r[...], hr_r[...]], axis=1)
        agg = jnp.concatenate([al_r[...], ar_r[...]], axis=1)
        z = h + agg + jnp.dot(a0_r[...] + a1_r[...], wp_r[...],
                              preferred_element_type=_f32)
        t = jnp.maximum(jnp.dot(z, w1_r[...], preferred_element_type=_f32)
                        + b1_r[...], 0.0)
        z2 = jnp.dot(t, w2_r[...], preferred_element_type=_f32) + b2_r[...]
        z2_r[...] = z2
        su = jnp.sum(z2, axis=0, keepdims=True)
        sq = jnp.sum(z2 * z2, axis=0, keepdims=True)
        upd = jnp.concatenate([su, sq, jnp.zeros((6, _D), _f32)], axis=0)
        prev = jnp.where(i == 0, jnp.zeros_like(upd), st_r[...])
        st_r[...] = prev + upd

    return pl.pallas_call(
        body,
        grid=(_G,),
        in_specs=[_row_spec(_HALF), _row_spec(_HALF),
                  _row_spec(_HALF), _row_spec(_HALF),
                  _row_spec(16), _row_spec(16),
                  _full_spec((16, _D)),
                  _full_spec((_D, 2 * _D)), _full_spec((1, 2 * _D)),
                  _full_spec((2 * _D, _D)), _full_spec((1, _D))],
        out_specs=[_row_spec(_D), _full_spec((8, _D))],
        out_shape=[jax.ShapeDtypeStruct((_N, _D), _f32),
                   jax.ShapeDtypeStruct((8, _D), _f32)],
    )(hl, hr, aggl, aggr, a0, a1, wp, w1, b1.reshape(1, -1), w2,
      b2.reshape(1, -1))


def _tc_bn(z2, stats, g, b, relu):
    def body(z_r, st_r, g_r, b_r, hl_r, hr_r):
        mu = st_r[0:1, :] / _N
        var = st_r[1:2, :] / _N - mu * mu
        inv = lax.rsqrt(var + 1e-5)
        y = (z_r[...] - mu) * (inv * g_r[...]) + b_r[...]
        if relu:
            y = jnp.maximum(y, 0.0)
        hl_r[...] = y[:, :_HALF]
        hr_r[...] = y[:, _HALF:]

    return pl.pallas_call(
        body,
        grid=(_G,),
        in_specs=[_row_spec(_D), _full_spec((8, _D)),
                  _full_spec((1, _D)), _full_spec((1, _D))],
        out_specs=[_row_spec(_HALF)] * 2,
        out_shape=[jax.ShapeDtypeStruct((_N, _HALF), _f32)] * 2,
    )(z2, stats, g.reshape(1, -1), b.reshape(1, -1))


def _tc_conv3(al, ar, c0, c1, iso, wrp, wri, wtp, wti, br):
    efgs = iso.shape[1]

    def body(al_r, ar_r, c0_r, c1_r, iso_r, wrp_r, wri_r, wtp_r, wti_r, br_r,
             yl_r, yr_r, r3_r):
        cnt = jnp.maximum(c0_r[...][:, 0:1] + c1_r[...][:, 0:1], 1.0)
        pooled = jnp.concatenate([al_r[...], ar_r[...]], axis=1) / cnt
        y = (jnp.dot(pooled, wrp_r[...], preferred_element_type=_f32)
             + jnp.dot(iso_r[...], wri_r[...], preferred_element_type=_f32))
        r3 = (jnp.dot(pooled, wtp_r[...], preferred_element_type=_f32)
              + jnp.dot(iso_r[...], wti_r[...], preferred_element_type=_f32)
              + br_r[...])
        yl_r[...] = y[:, :_HALF]
        yr_r[...] = y[:, _HALF:]
        r3_r[...] = r3

    return pl.pallas_call(
        body,
        grid=(_G,),
        in_specs=[_row_spec(_HALF), _row_spec(_HALF),
                  _row_spec(16), _row_spec(16), _row_spec(efgs),
                  _full_spec((_D, _D)), _full_spec((efgs, _D)),
                  _full_spec((_D, _D)), _full_spec((efgs, _D)),
                  _full_spec((1, _D))],
        out_specs=[_row_spec(_HALF), _row_spec(_HALF), _row_spec(_D)],
        out_shape=[jax.ShapeDtypeStruct((_N, _HALF), _f32),
                   jax.ShapeDtypeStruct((_N, _HALF), _f32),
                   jax.ShapeDtypeStruct((_N, _D), _f32)],
    )(al, ar, c0, c1, iso, wrp, wri, wtp, wti, br.reshape(1, -1))


def _tc_conv4(al, ar, r3, wrel, wroot, br):
    def body(al_r, ar_r, r3_r, wrel_r, wroot_r, br_r, yl_r, yr_r, r4_r):
        xc1 = jnp.maximum(jnp.concatenate([al_r[...], ar_r[...]], axis=1)
                          + r3_r[...], 0.0)
        y = jnp.dot(xc1, wrel_r[...], preferred_element_type=_f32)
        r4 = jnp.dot(xc1, wroot_r[...], preferred_element_type=_f32) + br_r[...]
        yl_r[...] = y[:, :_HALF]
        yr_r[...] = y[:, _HALF:]
        r4_r[...] = r4

    return pl.pallas_call(
        body,
        grid=(_G,),
        in_specs=[_row_spec(_HALF), _row_spec(_HALF), _row_spec(_D),
                  _full_spec((_D, _D)), _full_spec((_D, _D)),
                  _full_spec((1, _D))],
        out_specs=[_row_spec(_HALF), _row_spec(_HALF), _row_spec(_D)],
        out_shape=[jax.ShapeDtypeStruct((_N, _HALF), _f32),
                   jax.ShapeDtypeStruct((_N, _HALF), _f32),
                   jax.ShapeDtypeStruct((_N, _D), _f32)],
    )(al, ar, r3, wrel, wroot, br.reshape(1, -1))


def _tc_readout(nl, nr, al, ar, r4, bt3d, b33d,
                w0a, w0b, b0, w1, b1, w2, b2, w3, b3):
    def body(nl_r, nr_r, al_r, ar_r, r4_r, bt_r, b3_r,
             w0a_r, w0b_r, b0_r, w1_r, b1_r, w2_r, b2_r, w3_r, b3_r2,
             out_r, acc1, acc3, cnt1, cnt3):
        i = pl.program_id(0)

        @pl.when(i == 0)
        def _():
            acc1[...] = jnp.zeros_like(acc1)
            acc3[...] = jnp.zeros_like(acc3)
            cnt1[...] = jnp.zeros_like(cnt1)
            cnt3[...] = jnp.zeros_like(cnt3)

        nrep = jnp.concatenate([nl_r[...], nr_r[...]], axis=1)
        xc2 = jnp.maximum(jnp.concatenate([al_r[...], ar_r[...]], axis=1)
                          + r4_r[...], 0.0)
        ids = lax.broadcasted_iota(jnp.int32, (_B, _R), 0)
        oh1 = (bt_r[...].reshape(1, _R) == ids).astype(_f32)
        oh3 = (b3_r[...].reshape(1, _R) == ids).astype(_f32)
        acc1[...] += jnp.dot(oh1, nrep, preferred_element_type=_f32)
        acc3[...] += jnp.dot(oh3, xc2, preferred_element_type=_f32)
        cnt1[...] += jnp.sum(oh1, axis=1, keepdims=True)
        cnt3[...] += jnp.sum(oh3, axis=1, keepdims=True)

        @pl.when(i == _G - 1)
        def _():
            x1 = acc1[...] / jnp.maximum(cnt1[...], 1.0)
            x3 = acc3[...] / jnp.maximum(cnt3[...], 1.0)
            m = jnp.maximum(jnp.dot(x1, w0a_r[...], preferred_element_type=_f32)
                            + jnp.dot(x3, w0b_r[...], preferred_element_type=_f32)
                            + b0_r[...], 0.0)
            m = jnp.maximum(jnp.dot(m, w1_r[...], preferred_element_type=_f32)
                            + b1_r[...], 0.0)
            m = jnp.maximum(jnp.dot(m, w2_r[...], preferred_element_type=_f32)
                            + b2_r[...], 0.0)
            out_r[...] = (jnp.dot(m, w3_r[...], preferred_element_type=_f32)
                          + b3_r2[...])

    idx_spec = pl.BlockSpec((1, 1, _R), lambda i: (i, 0, 0))
    return pl.pallas_call(
        body,
        grid=(_G,),
        in_specs=[_row_spec(_HALF), _row_spec(_HALF),
                  _row_spec(_HALF), _row_spec(_HALF), _row_spec(_D),
                  idx_spec, idx_spec,
                  _full_spec((_D, _D)), _full_spec((_D, _D)),
                  _full_spec((1, _D)),
                  _full_spec((_D, _D // 2)), _full_spec((1, _D // 2)),
                  _full_spec((_D // 2, _D // 4)), _full_spec((1, _D // 4)),
                  _full_spec((_D // 4, 1)), _full_spec((1, 1))],
        out_specs=_full_spec((_B, 1)),
        out_shape=jax.ShapeDtypeStruct((_B, 1), _f32),
        scratch_shapes=[pltpu.VMEM((_B, _D), _f32), pltpu.VMEM((_B, _D), _f32),
                        pltpu.VMEM((_B, 1), _f32), pltpu.VMEM((_B, 1), _f32)],
    )(nl, nr, al, ar, r4, bt3d, b33d, w0a, w0b, b0.reshape(1, -1),
      w1, b1.reshape(1, -1), w2, b2.reshape(1, -1), w3, b3.reshape(1, -1))


def _pad_edges(src, dst, mult=4096):
    e = src.shape[0]
    ep = ((e + mult - 1) // mult) * mult
    src = jnp.concatenate([src, jnp.zeros((ep - e,), jnp.int32)])
    dst = jnp.concatenate([dst, jnp.full((ep - e,), _N, jnp.int32)])
    return src.reshape(-1, 128), dst.reshape(-1, 128)


def kernel(x, edge_index, edge_attr, batch, iso_type_3, edge_index_3,
           assignment_index_3, batch_3, W_emb, b_emb, W_edge, b_edge,
           W_mlp1, b_mlp1, W_mlp2, b_mlp2, bn_g, bn_b, W_rel3, b_rel3,
           W_root3, W_rel4, b_rel4, W_root4, W_out0, b_out0, W_out1,
           b_out1, W_out2, b_out2, W_out3, b_out3):
    src1, dst1 = _pad_edges(edge_index[0], edge_index[1])
    src3, dst3 = _pad_edges(edge_index_3[0], edge_index_3[1])
    srca, dsta = _pad_edges(assignment_index_3[0], assignment_index_3[1])
    e1 = edge_index.shape[1]
    ea_pad = jnp.zeros((src1.shape[0] * 128, 16), _f32)
    ea_pad = ea_pad.at[:e1, :4].set(edge_attr).at[:e1, 4].set(1.0)
    ones_src = jnp.zeros((128, 16), _f32).at[:, 0].set(1.0)
    zeros_l = jnp.zeros((_NACC, _HALF), _f32)
    zeros16 = jnp.zeros((_NACC, 16), _f32)

    a0, a1, c30, c31 = _counts_sc(ea_pad, dst1, ones_src, dsta, zeros16)

    hl, hr = _tc_emb(x, W_emb, b_emb)
    for l in range(3):
        wp = (jnp.zeros((16, _D), _f32)
              .at[:4].set(W_edge[l]).at[4].set(b_edge[l]))
        aggl, aggr = _seg_sum_sc(hl, hr, src1, dst1, zeros_l)
        z2, st = _tc_layer(hl, hr, aggl, aggr, a0, a1, wp,
                           W_mlp1[l], b_mlp1[l], W_mlp2[l], b_mlp2[l])
        hl, hr = _tc_bn(z2, st, bn_g[l], bn_b[l], relu=(l < 2))

    p3l, p3r = _seg_sum_sc(hl, hr, srca, dsta, zeros_l)
    y3l, y3r, r3 = _tc_conv3(p3l, p3r, c30, c31, iso_type_3,
                             W_rel3[:_D], W_rel3[_D:],
                             W_root3[:_D], W_root3[_D:], b_rel3)
    a3l, a3r = _seg_sum_sc(y3l, y3r, src3, dst3, zeros_l)
    y4l, y4r, r4 = _tc_conv4(a3l, a3r, r3, W_rel4, W_root4, b_rel4)
    a4l, a4r = _seg_sum_sc(y4l, y4r, src3, dst3, zeros_l)

    out = _tc_readout(hl, hr, a4l, a4r, r4,
                      batch.reshape(_G, 1, _R), batch_3.reshape(_G, 1, _R),
                      W_out0[:_D], W_out0[_D:], b_out0,
                      W_out1, b_out1, W_out2, b_out2, W_out3, b_out3)
    return out.reshape(-1)


# speedup vs baseline: 1.6415x; 1.6415x over previous
import functools

import jax
import jax.numpy as jnp
from jax import lax
from jax.experimental import pallas as pl
from jax.experimental.pallas import tpu as pltpu
from jax.experimental.pallas import tpu_sc as plsc

_N = 10000
_B = 256
_D = 256
_HALF = 128
_NACC = 10112
_RPT = 632
_R = 400
_G = _N // _R

_f32 = jnp.float32

_dot = functools.partial(jnp.dot, preferred_element_type=jnp.float32,
                         precision=jax.lax.Precision.HIGHEST)
_dotd = functools.partial(jnp.dot, preferred_element_type=jnp.float32)


def _sc_mesh():
    return plsc.VectorSubcoreMesh(core_axis_name="c", subcore_axis_name="s")


def _seg_sum_sc(tab_l, tab_r, src2d, dst2d, zeros_l):
    ep = src2d.shape[0]
    cpt = ep // 16

    @functools.partial(
        pl.kernel,
        out_type=[jax.ShapeDtypeStruct((_N, _HALF), _f32)] * 2,
        mesh=_sc_mesh(),
        scratch_types=[
            pltpu.VMEM((128,), jnp.int32),
            pltpu.VMEM((128,), jnp.int32),
            pltpu.VMEM((128, _HALF), _f32),
            pltpu.VMEM_SHARED((_NACC, _HALF), _f32),
            pltpu.SemaphoreType.DMA,
        ],
    )
    def k(tabl_h, tabr_h, src_h, dst_h, z_h, outl_h, outr_h,
          idxg, idxs, rows, acc, sem):
        c = lax.axis_index("c")
        s = lax.axis_index("s")
        r0 = s * _RPT
        pltpu.sync_copy(z_h.at[pl.ds(r0, _RPT)], acc.at[pl.ds(r0, _RPT)])
        plsc.subcore_barrier()

        def run(tab_h):
            def body(i, carry):
                row = s * cpt + i
                pltpu.sync_copy(src_h.at[row], idxg)
                pltpu.async_copy(tab_h.at[idxg], rows, sem).wait()
                pltpu.sync_copy(dst_h.at[row], idxs)
                pltpu.sync_copy(rows, acc.at[idxs], add=True)
                return carry
            lax.fori_loop(0, cpt, body, 0)

        @pl.when(c == 0)
        def _():
            run(tabl_h)

        @pl.when(c == 1)
        def _():
            run(tabr_h)

        plsc.subcore_barrier()
        rd = jnp.minimum(r0, _N - _RPT)

        @pl.when(c == 0)
        def _():
            pltpu.sync_copy(acc.at[pl.ds(rd, _RPT)], outl_h.at[pl.ds(rd, _RPT)])

        @pl.when(c == 1)
        def _():
            pltpu.sync_copy(acc.at[pl.ds(rd, _RPT)], outr_h.at[pl.ds(rd, _RPT)])

    return k(tab_l, tab_r, src2d, dst2d, zeros_l)


def _seg_sum_msg_sc(tab_l, tab_r, el, er, src2d, dst2d, zeros_l):
    ep = src2d.shape[0]
    cpt = ep // 16

    @functools.partial(
        pl.kernel,
        out_type=[jax.ShapeDtypeStruct((_N, _HALF), _f32)] * 2,
        mesh=_sc_mesh(),
        scratch_types=[
            pltpu.VMEM((128,), jnp.int32),
            pltpu.VMEM((128,), jnp.int32),
            pltpu.VMEM((128, _HALF), _f32),
            pltpu.VMEM((128, _HALF), _f32),
            pltpu.VMEM_SHARED((_NACC, _HALF), _f32),
            pltpu.SemaphoreType.DMA,
        ],
    )
    def k(tabl_h, tabr_h, el_h, er_h, src_h, dst_h, z_h, outl_h, outr_h,
          idxg, idxs, rows, rows2, acc, sem):
        c = lax.axis_index("c")
        s = lax.axis_index("s")
        r0 = s * _RPT
        pltpu.sync_copy(z_h.at[pl.ds(r0, _RPT)], acc.at[pl.ds(r0, _RPT)])
        plsc.subcore_barrier()

        def run(tab_h, e_h):
            def body(i, carry):
                row = s * cpt + i
                pltpu.sync_copy(src_h.at[row], idxg)
                pltpu.async_copy(tab_h.at[idxg], rows, sem).wait()
                pltpu.sync_copy(dst_h.at[row], idxs)
                pltpu.sync_copy(rows, acc.at[idxs], add=True)
                pltpu.async_copy(e_h.at[pl.ds(row * 128, 128)], rows2,
                                 sem).wait()
                pltpu.sync_copy(rows2, acc.at[idxs], add=True)
                return carry
            lax.fori_loop(0, cpt, body, 0)

        @pl.when(c == 0)
        def _():
            run(tabl_h, el_h)

        @pl.when(c == 1)
        def _():
            run(tabr_h, er_h)

        plsc.subcore_barrier()
        rd = jnp.minimum(r0, _N - _RPT)

        @pl.when(c == 0)
        def _():
            pltpu.sync_copy(acc.at[pl.ds(rd, _RPT)], outl_h.at[pl.ds(rd, _RPT)])

        @pl.when(c == 1)
        def _():
            pltpu.sync_copy(acc.at[pl.ds(rd, _RPT)], outr_h.at[pl.ds(rd, _RPT)])

    return k(tab_l, tab_r, el, er, src2d, dst2d, zeros_l)


def _cnt_sc(ones_src, col2d, zeros_l):
    epb = col2d.shape[0]
    half = epb // 2
    cpb = half // 16

    @functools.partial(
        pl.kernel,
        out_type=[jax.ShapeDtypeStruct((_N, _HALF), _f32)] * 2,
        mesh=_sc_mesh(),
        scratch_types=[
            pltpu.VMEM((128,), jnp.int32),
            pltpu.VMEM((128, _HALF), _f32),
            pltpu.VMEM_SHARED((_NACC, _HALF), _f32),
        ],
    )
    def k(ones_h, col_h, z_h, c0_h, c1_h, idxs, rows, acc):
        c = lax.axis_index("c")
        s = lax.axis_index("s")
        r0 = s * _RPT
        pltpu.sync_copy(z_h.at[pl.ds(r0, _RPT)], acc.at[pl.ds(r0, _RPT)])
        plsc.subcore_barrier()
        pltpu.sync_copy(ones_h, rows)

        def body(i, carry):
            row = c * half + s * cpb + i
            pltpu.sync_copy(col_h.at[row], idxs)
            pltpu.sync_copy(rows, acc.at[idxs], add=True)
            return carry
        lax.fori_loop(0, cpb, body, 0)

        plsc.subcore_barrier()
        rd = jnp.minimum(r0, _N - _RPT)

        @pl.when(c == 0)
        def _():
            pltpu.sync_copy(acc.at[pl.ds(rd, _RPT)], c0_h.at[pl.ds(rd, _RPT)])

        @pl.when(c == 1)
        def _():
            pltpu.sync_copy(acc.at[pl.ds(rd, _RPT)], c1_h.at[pl.ds(rd, _RPT)])

    return k(ones_src, col2d, zeros_l)


def _row_spec(width):
    return pl.BlockSpec((_R, width), lambda i: (i, 0))


def _full_spec(shape):
    nd = len(shape)
    return pl.BlockSpec(shape, lambda i: (0,) * nd)


def _tc_emb(x, w, b):
    d_in = x.shape[1]

    def body(x_r, w_r, b_r, hl_r, hr_r):
        z = _dotd(x_r[...], w_r[...]) + b_r[...]
        z = jnp.maximum(z, 0.0)
        hl_r[...] = z[:, :_HALF]
        hr_r[...] = z[:, _HALF:]

    return pl.pallas_call(
        body,
        grid=(_G,),
        in_specs=[_row_spec(d_in), _full_spec((d_in, _D)), _full_spec((1, _D))],
        out_specs=[_row_spec(_HALF)] * 2,
        out_shape=[jax.ShapeDtypeStruct((_N, _HALF), _f32)] * 2,
    )(x, w, b.reshape(1, -1))


def _tc_eproj(ea8, wp8):
    ep = ea8.shape[0]
    re = 4096
    ge = ep // re
    spec = pl.BlockSpec((re, 8), lambda i: (i, 0))
    ospec = pl.BlockSpec((re, _HALF), lambda i: (i, 0))

    def body(ea_r, wp_r, el_r, er_r):
        e = _dotd(ea_r[...], wp_r[...])
        el_r[...] = e[:, :_HALF]
        er_r[...] = e[:, _HALF:]

    return pl.pallas_call(
        body,
        grid=(ge,),
        in_specs=[spec, _full_spec((8, _D))],
        out_specs=[ospec, ospec],
        out_shape=[jax.ShapeDtypeStruct((ep, _HALF), _f32)] * 2,
    )(ea8, wp8)


def _tc_layer(hl, hr, aggl, aggr, w1, b1, w2, b2):
    def body(hl_r, hr_r, al_r, ar_r, w1_r, b1_r, w2_r, b2_r,
             z2_r, st_r):
        i = pl.program_id(0)
        h = jnp.concatenate([hl_r[...], hr_r[...]], axis=1)
        agg = jnp.concatenate([al_r[...], ar_r[...]], axis=1)
        z = h + agg
        t = jnp.maximum(_dotd(z, w1_r[...])
                        + b1_r[...], 0.0)
        z2 = _dotd(t, w2_r[...]) + b2_r[...]
        z2_r[...] = z2
        su = jnp.sum(z2, axis=0, keepdims=True)
        sq = jnp.sum(z2 * z2, axis=0, keepdims=True)
        upd = jnp.concatenate([su, sq, jnp.zeros((6, _D), _f32)], axis=0)
        prev = jnp.where(i == 0, jnp.zeros_like(upd), st_r[...])
        st_r[...] = prev + upd

    return pl.pallas_call(
        body,
        grid=(_G,),
        in_specs=[_row_spec(_HALF), _row_spec(_HALF),
                  _row_spec(_HALF), _row_spec(_HALF),
                  _full_spec((_D, 2 * _D)), _full_spec((1, 2 * _D)),
                  _full_spec((2 * _D, _D)), _full_spec((1, _D))],
        out_specs=[_row_spec(_D), _full_spec((8, _D))],
        out_shape=[jax.ShapeDtypeStruct((_N, _D), _f32),
                   jax.ShapeDtypeStruct((8, _D), _f32)],
    )(hl, hr, aggl, aggr, w1, b1.reshape(1, -1), w2,
      b2.reshape(1, -1))


def _tc_bn(z2, stats, g, b, relu):
    def body(z_r, st_r, g_r, b_r, hl_r, hr_r):
        mu = st_r[0:1, :] / _N
        var = st_r[1:2, :] / _N - mu * mu
        y = (z_r[...] - mu) / jnp.sqrt(var + 1e-5) * g_r[...] + b_r[...]
        if relu:
            y = jnp.maximum(y, 0.0)
        hl_r[...] = y[:, :_HALF]
        hr_r[...] = y[:, _HALF:]

    return pl.pallas_call(
        body,
        grid=(_G,),
        in_specs=[_row_spec(_D), _full_spec((8, _D)),
                  _full_spec((1, _D)), _full_spec((1, _D))],
        out_specs=[_row_spec(_HALF)] * 2,
        out_shape=[jax.ShapeDtypeStruct((_N, _HALF), _f32)] * 2,
    )(z2, stats, g.reshape(1, -1), b.reshape(1, -1))


def _tc_conv3(al, ar, c0, c1, iso, wrp, wri, wtp, wti, br):
    efgs = iso.shape[1]

    def body(al_r, ar_r, c0_r, c1_r, iso_r, wrp_r, wri_r, wtp_r, wti_r, br_r,
             yl_r, yr_r, r3_r):
        cnt = jnp.maximum(c0_r[...][:, 0:1] + c1_r[...][:, 0:1], 1.0)
        pooled = jnp.concatenate([al_r[...], ar_r[...]], axis=1) / cnt
        y = (_dotd(pooled, wrp_r[...])
             + _dotd(iso_r[...], wri_r[...]))
        r3 = (_dotd(pooled, wtp_r[...])
              + _dotd(iso_r[...], wti_r[...])
              + br_r[...])
        yl_r[...] = y[:, :_HALF]
        yr_r[...] = y[:, _HALF:]
        r3_r[...] = r3

    return pl.pallas_call(
        body,
        grid=(_G,),
        in_specs=[_row_spec(_HALF), _row_spec(_HALF),
                  _row_spec(_HALF), _row_spec(_HALF), _row_spec(efgs),
                  _full_spec((_D, _D)), _full_spec((efgs, _D)),
                  _full_spec((_D, _D)), _full_spec((efgs, _D)),
                  _full_spec((1, _D))],
        out_specs=[_row_spec(_HALF), _row_spec(_HALF), _row_spec(_D)],
        out_shape=[jax.ShapeDtypeStruct((_N, _HALF), _f32),
                   jax.ShapeDtypeStruct((_N, _HALF), _f32),
                   jax.ShapeDtypeStruct((_N, _D), _f32)],
    )(al, ar, c0, c1, iso, wrp, wri, wtp, wti, br.reshape(1, -1))


def _tc_conv4(al, ar, r3, wrel, wroot, br):
    def body(al_r, ar_r, r3_r, wrel_r, wroot_r, br_r, yl_r, yr_r, r4_r):
        xc1 = jnp.maximum(jnp.concatenate([al_r[...], ar_r[...]], axis=1)
                          + r3_r[...], 0.0)
        y = _dotd(xc1, wrel_r[...])
        r4 = _dotd(xc1, wroot_r[...]) + br_r[...]
        yl_r[...] = y[:, :_HALF]
        yr_r[...] = y[:, _HALF:]
        r4_r[...] = r4

    return pl.pallas_call(
        body,
        grid=(_G,),
        in_specs=[_row_spec(_HALF), _row_spec(_HALF), _row_spec(_D),
                  _full_spec((_D, _D)), _full_spec((_D, _D)),
                  _full_spec((1, _D))],
        out_specs=[_row_spec(_HALF), _row_spec(_HALF), _row_spec(_D)],
        out_shape=[jax.ShapeDtypeStruct((_N, _HALF), _f32),
                   jax.ShapeDtypeStruct((_N, _HALF), _f32),
                   jax.ShapeDtypeStruct((_N, _D), _f32)],
    )(al, ar, r3, wrel, wroot, br.reshape(1, -1))


def _tc_readout(nl, nr, al, ar, r4, bt3d, b33d,
                w0a, w0b, b0, w1, b1, w2, b2, w3, b3):
    def body(nl_r, nr_r, al_r, ar_r, r4_r, bt_r, b3_r,
             w0a_r, w0b_r, b0_r, w1_r, b1_r, w2_r, b2_r, w3_r, b3_r2,
             out_r, acc1, acc3, cnt1, cnt3):
        i = pl.program_id(0)

        @pl.when(i == 0)
        def _():
            acc1[...] = jnp.zeros_like(acc1)
            acc3[...] = jnp.zeros_like(acc3)
            cnt1[...] = jnp.zeros_like(cnt1)
            cnt3[...] = jnp.zeros_like(cnt3)

        nrep = jnp.concatenate([nl_r[...], nr_r[...]], axis=1)
        xc2 = jnp.maximum(jnp.concatenate([al_r[...], ar_r[...]], axis=1)
                          + r4_r[...], 0.0)
        ids = lax.broadcasted_iota(jnp.int32, (_B, _R), 0)
        oh1 = (bt_r[...].reshape(1, _R) == ids).astype(_f32)
        oh3 = (b3_r[...].reshape(1, _R) == ids).astype(_f32)
        acc1[...] += _dot(oh1, nrep)
        acc3[...] += _dot(oh3, xc2)
        cnt1[...] += jnp.sum(oh1, axis=1, keepdims=True)
        cnt3[...] += jnp.sum(oh3, axis=1, keepdims=True)

        @pl.when(i == _G - 1)
        def _():
            x1 = acc1[...] / jnp.maximum(cnt1[...], 1.0)
            x3 = acc3[...] / jnp.maximum(cnt3[...], 1.0)
            m = jnp.maximum(_dotd(x1, w0a_r[...])
                            + _dotd(x3, w0b_r[...])
                            + b0_r[...], 0.0)
            m = jnp.maximum(_dotd(m, w1_r[...])
                            + b1_r[...], 0.0)
            m = jnp.maximum(_dotd(m, w2_r[...])
                            + b2_r[...], 0.0)
            out_r[...] = (_dotd(m, w3_r[...])
                          + b3_r2[...])

    idx_spec = pl.BlockSpec((1, 1, _R), lambda i: (i, 0, 0))
    return pl.pallas_call(
        body,
        grid=(_G,),
        in_specs=[_row_spec(_HALF), _row_spec(_HALF),
                  _row_spec(_HALF), _row_spec(_HALF), _row_spec(_D),
                  idx_spec, idx_spec,
                  _full_spec((_D, _D)), _full_spec((_D, _D)),
                  _full_spec((1, _D)),
                  _full_spec((_D, _D // 2)), _full_spec((1, _D // 2)),
                  _full_spec((_D // 2, _D // 4)), _full_spec((1, _D // 4)),
                  _full_spec((_D // 4, 1)), _full_spec((1, 1))],
        out_specs=_full_spec((_B, 1)),
        out_shape=jax.ShapeDtypeStruct((_B, 1), _f32),
        scratch_shapes=[pltpu.VMEM((_B, _D), _f32), pltpu.VMEM((_B, _D), _f32),
                        pltpu.VMEM((_B, 1), _f32), pltpu.VMEM((_B, 1), _f32)],
    )(nl, nr, al, ar, r4, bt3d, b33d, w0a, w0b, b0.reshape(1, -1),
      w1, b1.reshape(1, -1), w2, b2.reshape(1, -1), w3, b3.reshape(1, -1))


def _pad_edges(src, dst, mult=4096):
    e = src.shape[0]
    ep = ((e + mult - 1) // mult) * mult
    src = jnp.concatenate([src, jnp.zeros((ep - e,), jnp.int32)])
    dst = jnp.concatenate([dst, jnp.full((ep - e,), _N, jnp.int32)])
    return src.reshape(-1, 128), dst.reshape(-1, 128)


def kernel(x, edge_index, edge_attr, batch, iso_type_3, edge_index_3,
           assignment_index_3, batch_3, W_emb, b_emb, W_edge, b_edge,
           W_mlp1, b_mlp1, W_mlp2, b_mlp2, bn_g, bn_b, W_rel3, b_rel3,
           W_root3, W_rel4, b_rel4, W_root4, W_out0, b_out0, W_out1,
           b_out1, W_out2, b_out2, W_out3, b_out3):
    src1, dst1 = _pad_edges(edge_index[0], edge_index[1])
    src3, dst3 = _pad_edges(edge_index_3[0], edge_index_3[1])
    srca, dsta = _pad_edges(assignment_index_3[0], assignment_index_3[1])
    e1 = edge_index.shape[1]
    ea8 = jnp.zeros((src1.shape[0] * 128, 8), _f32)
    ea8 = ea8.at[:e1, :4].set(edge_attr).at[:e1, 4].set(1.0)
    ones_src = jnp.zeros((128, _HALF), _f32).at[:, 0].set(1.0)
    zeros_l = jnp.zeros((_NACC, _HALF), _f32)

    c30, c31 = _cnt_sc(ones_src, dsta, zeros_l)

    hl, hr = _tc_emb(x, W_emb, b_emb)
    for l in range(3):
        wp8 = (jnp.zeros((8, _D), _f32)
               .at[:4].set(W_edge[l]).at[4].set(b_edge[l]))
        el, er = _tc_eproj(ea8, wp8)
        aggl, aggr = _seg_sum_msg_sc(hl, hr, el, er, src1, dst1, zeros_l)
        z2, st = _tc_layer(hl, hr, aggl, aggr,
                           W_mlp1[l], b_mlp1[l], W_mlp2[l], b_mlp2[l])
        hl, hr = _tc_bn(z2, st, bn_g[l], bn_b[l], relu=(l < 2))

    p3l, p3r = _seg_sum_sc(hl, hr, srca, dsta, zeros_l)
    y3l, y3r, r3 = _tc_conv3(p3l, p3r, c30, c31, iso_type_3,
                             W_rel3[:_D], W_rel3[_D:],
                             W_root3[:_D], W_root3[_D:], b_rel3)
    a3l, a3r = _seg_sum_sc(y3l, y3r, src3, dst3, zeros_l)
    y4l, y4r, r4 = _tc_conv4(a3l, a3r, r3, W_rel4, W_root4, b_rel4)
    a4l, a4r = _seg_sum_sc(y4l, y4r, src3, dst3, zeros_l)

    out = _tc_readout(hl, hr, a4l, a4r, r4,
                      batch.reshape(_G, 1, _R), batch_3.reshape(_G, 1, _R),
                      W_out0[:_D], W_out0[_D:], b_out0,
                      W_out1, b_out1, W_out2, b_out2, W_out3, b_out3)
    return out.reshape(-1)


# speedup vs baseline: 1.7998x; 1.0965x over previous
import functools

import jax
import jax.numpy as jnp
from jax import lax
from jax.experimental import pallas as pl
from jax.experimental.pallas import tpu as pltpu
from jax.experimental.pallas import tpu_sc as plsc

_N = 10000
_B = 256
_D = 256
_HALF = 128
_NACC = 10112
_RPT = 632
_R = 400
_G = _N // _R

_f32 = jnp.float32

_dot = functools.partial(jnp.dot, preferred_element_type=jnp.float32,
                         precision=jax.lax.Precision.HIGHEST)
_dotd = functools.partial(jnp.dot, preferred_element_type=jnp.float32)


def _sc_mesh():
    return plsc.VectorSubcoreMesh(core_axis_name="c", subcore_axis_name="s")


def _seg_sum_sc(tab_l, tab_r, src2d, dst2d, zeros_l):
    ep = src2d.shape[0]
    cpt = ep // 16

    @functools.partial(
        pl.kernel,
        out_type=[jax.ShapeDtypeStruct((_N, _HALF), _f32)] * 2,
        mesh=_sc_mesh(),
        scratch_types=[
            pltpu.VMEM((128,), jnp.int32),
            pltpu.VMEM((128,), jnp.int32),
            pltpu.VMEM((128, _HALF), _f32),
            pltpu.VMEM_SHARED((_NACC, _HALF), _f32),
            pltpu.SemaphoreType.DMA,
        ],
    )
    def k(tabl_h, tabr_h, src_h, dst_h, z_h, outl_h, outr_h,
          idxg, idxs, rows, acc, sem):
        c = lax.axis_index("c")
        s = lax.axis_index("s")
        r0 = s * _RPT
        pltpu.sync_copy(z_h.at[pl.ds(r0, _RPT)], acc.at[pl.ds(r0, _RPT)])
        plsc.subcore_barrier()

        def run(tab_h):
            def body(i, carry):
                row = s * cpt + i
                pltpu.sync_copy(src_h.at[row], idxg)
                pltpu.async_copy(tab_h.at[idxg], rows, sem).wait()
                pltpu.sync_copy(dst_h.at[row], idxs)
                pltpu.sync_copy(rows, acc.at[idxs], add=True)
                return carry
            lax.fori_loop(0, cpt, body, 0)

        @pl.when(c == 0)
        def _():
            run(tabl_h)

        @pl.when(c == 1)
        def _():
            run(tabr_h)

        plsc.subcore_barrier()
        rd = jnp.minimum(r0, _N - _RPT)

        @pl.when(c == 0)
        def _():
            pltpu.sync_copy(acc.at[pl.ds(rd, _RPT)], outl_h.at[pl.ds(rd, _RPT)])

        @pl.when(c == 1)
        def _():
            pltpu.sync_copy(acc.at[pl.ds(rd, _RPT)], outr_h.at[pl.ds(rd, _RPT)])

    return k(tab_l, tab_r, src2d, dst2d, zeros_l)


def _seg_sum_msg_sc(tab_l, tab_r, el, er, src2d, dst2d, zeros_l):
    ep = src2d.shape[0]
    cpt = ep // 16

    @functools.partial(
        pl.kernel,
        out_type=[jax.ShapeDtypeStruct((_N, _HALF), _f32)] * 2,
        mesh=_sc_mesh(),
        scratch_types=[
            pltpu.VMEM((128,), jnp.int32),
            pltpu.VMEM((128,), jnp.int32),
            pltpu.VMEM((128, _HALF), _f32),
            pltpu.VMEM((128, _HALF), _f32),
            pltpu.VMEM_SHARED((_NACC, _HALF), _f32),
            pltpu.SemaphoreType.DMA,
            pltpu.SemaphoreType.DMA,
        ],
    )
    def k(tabl_h, tabr_h, el_h, er_h, src_h, dst_h, z_h, outl_h, outr_h,
          idxg, idxs, rows, rows2, acc, sem, sem2):
        c = lax.axis_index("c")
        s = lax.axis_index("s")
        r0 = s * _RPT
        pltpu.sync_copy(z_h.at[pl.ds(r0, _RPT)], acc.at[pl.ds(r0, _RPT)])
        plsc.subcore_barrier()

        def run(tab_h, e_h):
            def body(i, carry):
                row = s * cpt + i
                pltpu.sync_copy(src_h.at[row], idxg)
                cp_h = pltpu.async_copy(tab_h.at[idxg], rows, sem)
                cp_e = pltpu.async_copy(e_h.at[pl.ds(row * 128, 128)], rows2,
                                        sem2)
                pltpu.sync_copy(dst_h.at[row], idxs)
                cp_h.wait()
                pltpu.sync_copy(rows, acc.at[idxs], add=True)
                cp_e.wait()
                pltpu.sync_copy(rows2, acc.at[idxs], add=True)
                return carry
            lax.fori_loop(0, cpt, body, 0)

        @pl.when(c == 0)
        def _():
            run(tabl_h, el_h)

        @pl.when(c == 1)
        def _():
            run(tabr_h, er_h)

        plsc.subcore_barrier()
        rd = jnp.minimum(r0, _N - _RPT)

        @pl.when(c == 0)
        def _():
            pltpu.sync_copy(acc.at[pl.ds(rd, _RPT)], outl_h.at[pl.ds(rd, _RPT)])

        @pl.when(c == 1)
        def _():
            pltpu.sync_copy(acc.at[pl.ds(rd, _RPT)], outr_h.at[pl.ds(rd, _RPT)])

    return k(tab_l, tab_r, el, er, src2d, dst2d, zeros_l)


def _cnt_sc(ones_src, col2d, zeros_l):
    epb = col2d.shape[0]
    half = epb // 2
    cpb = half // 16

    @functools.partial(
        pl.kernel,
        out_type=[jax.ShapeDtypeStruct((_N, _HALF), _f32)] * 2,
        mesh=_sc_mesh(),
        scratch_types=[
            pltpu.VMEM((128,), jnp.int32),
            pltpu.VMEM((128, _HALF), _f32),
            pltpu.VMEM_SHARED((_NACC, _HALF), _f32),
        ],
    )
    def k(ones_h, col_h, z_h, c0_h, c1_h, idxs, rows, acc):
        c = lax.axis_index("c")
        s = lax.axis_index("s")
        r0 = s * _RPT
        pltpu.sync_copy(z_h.at[pl.ds(r0, _RPT)], acc.at[pl.ds(r0, _RPT)])
        plsc.subcore_barrier()
        pltpu.sync_copy(ones_h, rows)

        def body(i, carry):
            row = c * half + s * cpb + i
            pltpu.sync_copy(col_h.at[row], idxs)
            pltpu.sync_copy(rows, acc.at[idxs], add=True)
            return carry
        lax.fori_loop(0, cpb, body, 0)

        plsc.subcore_barrier()
        rd = jnp.minimum(r0, _N - _RPT)

        @pl.when(c == 0)
        def _():
            pltpu.sync_copy(acc.at[pl.ds(rd, _RPT)], c0_h.at[pl.ds(rd, _RPT)])

        @pl.when(c == 1)
        def _():
            pltpu.sync_copy(acc.at[pl.ds(rd, _RPT)], c1_h.at[pl.ds(rd, _RPT)])

    return k(ones_src, col2d, zeros_l)


def _row_spec(width):
    return pl.BlockSpec((_R, width), lambda i: (i, 0))


def _full_spec(shape):
    nd = len(shape)
    return pl.BlockSpec(shape, lambda i: (0,) * nd)


def _tc_emb(x, w, b):
    d_in = x.shape[1]

    def body(x_r, w_r, b_r, hl_r, hr_r):
        z = _dotd(x_r[...], w_r[...]) + b_r[...]
        z = jnp.maximum(z, 0.0)
        hl_r[...] = z[:, :_HALF]
        hr_r[...] = z[:, _HALF:]

    return pl.pallas_call(
        body,
        grid=(_G,),
        in_specs=[_row_spec(d_in), _full_spec((d_in, _D)), _full_spec((1, _D))],
        out_specs=[_row_spec(_HALF)] * 2,
        out_shape=[jax.ShapeDtypeStruct((_N, _HALF), _f32)] * 2,
    )(x, w, b.reshape(1, -1))


def _tc_eproj(ea8, wp8):
    ep = ea8.shape[0]
    re = 4096
    ge = ep // re
    spec = pl.BlockSpec((re, 8), lambda i: (i, 0))
    ospec = pl.BlockSpec((re, _HALF), lambda i: (i, 0))

    def body(ea_r, wp_r, el_r, er_r):
        e = _dotd(ea_r[...], wp_r[...])
        el_r[...] = e[:, :_HALF]
        er_r[...] = e[:, _HALF:]

    return pl.pallas_call(
        body,
        grid=(ge,),
        in_specs=[spec, _full_spec((8, _D))],
        out_specs=[ospec, ospec],
        out_shape=[jax.ShapeDtypeStruct((ep, _HALF), _f32)] * 2,
    )(ea8, wp8)


def _tc_layer(hl, hr, aggl, aggr, w1, b1, w2, b2):
    def body(hl_r, hr_r, al_r, ar_r, w1_r, b1_r, w2_r, b2_r,
             z2_r, st_r):
        i = pl.program_id(0)
        h = jnp.concatenate([hl_r[...], hr_r[...]], axis=1)
        agg = jnp.concatenate([al_r[...], ar_r[...]], axis=1)
        z = h + agg
        t = jnp.maximum(_dotd(z, w1_r[...])
                        + b1_r[...], 0.0)
        z2 = _dotd(t, w2_r[...]) + b2_r[...]
        z2_r[...] = z2
        su = jnp.sum(z2, axis=0, keepdims=True)
        sq = jnp.sum(z2 * z2, axis=0, keepdims=True)
        upd = jnp.concatenate([su, sq, jnp.zeros((6, _D), _f32)], axis=0)
        prev = jnp.where(i == 0, jnp.zeros_like(upd), st_r[...])
        st_r[...] = prev + upd

    return pl.pallas_call(
        body,
        grid=(_G,),
        in_specs=[_row_spec(_HALF), _row_spec(_HALF),
                  _row_spec(_HALF), _row_spec(_HALF),
                  _full_spec((_D, 2 * _D)), _full_spec((1, 2 * _D)),
                  _full_spec((2 * _D, _D)), _full_spec((1, _D))],
        out_specs=[_row_spec(_D), _full_spec((8, _D))],
        out_shape=[jax.ShapeDtypeStruct((_N, _D), _f32),
                   jax.ShapeDtypeStruct((8, _D), _f32)],
    )(hl, hr, aggl, aggr, w1, b1.reshape(1, -1), w2,
      b2.reshape(1, -1))


def _tc_bn(z2, stats, g, b, relu):
    def body(z_r, st_r, g_r, b_r, hl_r, hr_r):
        mu = st_r[0:1, :] / _N
        var = st_r[1:2, :] / _N - mu * mu
        y = (z_r[...] - mu) / jnp.sqrt(var + 1e-5) * g_r[...] + b_r[...]
        if relu:
            y = jnp.maximum(y, 0.0)
        hl_r[...] = y[:, :_HALF]
        hr_r[...] = y[:, _HALF:]

    return pl.pallas_call(
        body,
        grid=(_G,),
        in_specs=[_row_spec(_D), _full_spec((8, _D)),
                  _full_spec((1, _D)), _full_spec((1, _D))],
        out_specs=[_row_spec(_HALF)] * 2,
        out_shape=[jax.ShapeDtypeStruct((_N, _HALF), _f32)] * 2,
    )(z2, stats, g.reshape(1, -1), b.reshape(1, -1))


def _tc_conv3(al, ar, c0, c1, iso, wrp, wri, wtp, wti, br):
    efgs = iso.shape[1]

    def body(al_r, ar_r, c0_r, c1_r, iso_r, wrp_r, wri_r, wtp_r, wti_r, br_r,
             yl_r, yr_r, r3_r):
        cnt = jnp.maximum(c0_r[...][:, 0:1] + c1_r[...][:, 0:1], 1.0)
        pooled = jnp.concatenate([al_r[...], ar_r[...]], axis=1) / cnt
        y = (_dotd(pooled, wrp_r[...])
             + _dotd(iso_r[...], wri_r[...]))
        r3 = (_dotd(pooled, wtp_r[...])
              + _dotd(iso_r[...], wti_r[...])
              + br_r[...])
        yl_r[...] = y[:, :_HALF]
        yr_r[...] = y[:, _HALF:]
        r3_r[...] = r3

    return pl.pallas_call(
        body,
        grid=(_G,),
        in_specs=[_row_spec(_HALF), _row_spec(_HALF),
                  _row_spec(_HALF), _row_spec(_HALF), _row_spec(efgs),
                  _full_spec((_D, _D)), _full_spec((efgs, _D)),
                  _full_spec((_D, _D)), _full_spec((efgs, _D)),
                  _full_spec((1, _D))],
        out_specs=[_row_spec(_HALF), _row_spec(_HALF), _row_spec(_D)],
        out_shape=[jax.ShapeDtypeStruct((_N, _HALF), _f32),
                   jax.ShapeDtypeStruct((_N, _HALF), _f32),
                   jax.ShapeDtypeStruct((_N, _D), _f32)],
    )(al, ar, c0, c1, iso, wrp, wri, wtp, wti, br.reshape(1, -1))


def _tc_conv4(al, ar, r3, wrel, wroot, br):
    def body(al_r, ar_r, r3_r, wrel_r, wroot_r, br_r, yl_r, yr_r, r4_r):
        xc1 = jnp.maximum(jnp.concatenate([al_r[...], ar_r[...]], axis=1)
                          + r3_r[...], 0.0)
        y = _dotd(xc1, wrel_r[...])
        r4 = _dotd(xc1, wroot_r[...]) + br_r[...]
        yl_r[...] = y[:, :_HALF]
        yr_r[...] = y[:, _HALF:]
        r4_r[...] = r4

    return pl.pallas_call(
        body,
        grid=(_G,),
        in_specs=[_row_spec(_HALF), _row_spec(_HALF), _row_spec(_D),
                  _full_spec((_D, _D)), _full_spec((_D, _D)),
                  _full_spec((1, _D))],
        out_specs=[_row_spec(_HALF), _row_spec(_HALF), _row_spec(_D)],
        out_shape=[jax.ShapeDtypeStruct((_N, _HALF), _f32),
                   jax.ShapeDtypeStruct((_N, _HALF), _f32),
                   jax.ShapeDtypeStruct((_N, _D), _f32)],
    )(al, ar, r3, wrel, wroot, br.reshape(1, -1))


def _tc_readout(nl, nr, al, ar, r4, bt3d, b33d,
                w0a, w0b, b0, w1, b1, w2, b2, w3, b3):
    def body(nl_r, nr_r, al_r, ar_r, r4_r, bt_r, b3_r,
             w0a_r, w0b_r, b0_r, w1_r, b1_r, w2_r, b2_r, w3_r, b3_r2,
             out_r, acc1, acc3, cnt1, cnt3):
        i = pl.program_id(0)

        @pl.when(i == 0)
        def _():
            acc1[...] = jnp.zeros_like(acc1)
            acc3[...] = jnp.zeros_like(acc3)
            cnt1[...] = jnp.zeros_like(cnt1)
            cnt3[...] = jnp.zeros_like(cnt3)

        nrep = jnp.concatenate([nl_r[...], nr_r[...]], axis=1)
        xc2 = jnp.maximum(jnp.concatenate([al_r[...], ar_r[...]], axis=1)
                          + r4_r[...], 0.0)
        ids = lax.broadcasted_iota(jnp.int32, (_B, _R), 0)
        oh1 = (bt_r[...].reshape(1, _R) == ids).astype(_f32)
        oh3 = (b3_r[...].reshape(1, _R) == ids).astype(_f32)
        acc1[...] += _dot(oh1, nrep)
        acc3[...] += _dot(oh3, xc2)
        cnt1[...] += jnp.sum(oh1, axis=1, keepdims=True)
        cnt3[...] += jnp.sum(oh3, axis=1, keepdims=True)

        @pl.when(i == _G - 1)
        def _():
            x1 = acc1[...] / jnp.maximum(cnt1[...], 1.0)
            x3 = acc3[...] / jnp.maximum(cnt3[...], 1.0)
            m = jnp.maximum(_dotd(x1, w0a_r[...])
                            + _dotd(x3, w0b_r[...])
                            + b0_r[...], 0.0)
            m = jnp.maximum(_dotd(m, w1_r[...])
                            + b1_r[...], 0.0)
            m = jnp.maximum(_dotd(m, w2_r[...])
                            + b2_r[...], 0.0)
            out_r[...] = (_dotd(m, w3_r[...])
                          + b3_r2[...])

    idx_spec = pl.BlockSpec((1, 1, _R), lambda i: (i, 0, 0))
    return pl.pallas_call(
        body,
        grid=(_G,),
        in_specs=[_row_spec(_HALF), _row_spec(_HALF),
                  _row_spec(_HALF), _row_spec(_HALF), _row_spec(_D),
                  idx_spec, idx_spec,
                  _full_spec((_D, _D)), _full_spec((_D, _D)),
                  _full_spec((1, _D)),
                  _full_spec((_D, _D // 2)), _full_spec((1, _D // 2)),
                  _full_spec((_D // 2, _D // 4)), _full_spec((1, _D // 4)),
                  _full_spec((_D // 4, 1)), _full_spec((1, 1))],
        out_specs=_full_spec((_B, 1)),
        out_shape=jax.ShapeDtypeStruct((_B, 1), _f32),
        scratch_shapes=[pltpu.VMEM((_B, _D), _f32), pltpu.VMEM((_B, _D), _f32),
                        pltpu.VMEM((_B, 1), _f32), pltpu.VMEM((_B, 1), _f32)],
    )(nl, nr, al, ar, r4, bt3d, b33d, w0a, w0b, b0.reshape(1, -1),
      w1, b1.reshape(1, -1), w2, b2.reshape(1, -1), w3, b3.reshape(1, -1))


def _pad_edges(src, dst, mult=4096):
    e = src.shape[0]
    ep = ((e + mult - 1) // mult) * mult
    src = jnp.concatenate([src, jnp.zeros((ep - e,), jnp.int32)])
    dst = jnp.concatenate([dst, jnp.full((ep - e,), _N, jnp.int32)])
    return src.reshape(-1, 128), dst.reshape(-1, 128)


def kernel(x, edge_index, edge_attr, batch, iso_type_3, edge_index_3,
           assignment_index_3, batch_3, W_emb, b_emb, W_edge, b_edge,
           W_mlp1, b_mlp1, W_mlp2, b_mlp2, bn_g, bn_b, W_rel3, b_rel3,
           W_root3, W_rel4, b_rel4, W_root4, W_out0, b_out0, W_out1,
           b_out1, W_out2, b_out2, W_out3, b_out3):
    src1, dst1 = _pad_edges(edge_index[0], edge_index[1])
    src3, dst3 = _pad_edges(edge_index_3[0], edge_index_3[1])
    srca, dsta = _pad_edges(assignment_index_3[0], assignment_index_3[1])
    e1 = edge_index.shape[1]
    ea8 = jnp.zeros((src1.shape[0] * 128, 8), _f32)
    ea8 = ea8.at[:e1, :4].set(edge_attr).at[:e1, 4].set(1.0)
    ones_src = jnp.zeros((128, _HALF), _f32).at[:, 0].set(1.0)
    zeros_l = jnp.zeros((_NACC, _HALF), _f32)

    c30, c31 = _cnt_sc(ones_src, dsta, zeros_l)

    hl, hr = _tc_emb(x, W_emb, b_emb)
    for l in range(3):
        wp8 = (jnp.zeros((8, _D), _f32)
               .at[:4].set(W_edge[l]).at[4].set(b_edge[l]))
        el, er = _tc_eproj(ea8, wp8)
        aggl, aggr = _seg_sum_msg_sc(hl, hr, el, er, src1, dst1, zeros_l)
        z2, st = _tc_layer(hl, hr, aggl, aggr,
                           W_mlp1[l], b_mlp1[l], W_mlp2[l], b_mlp2[l])
        hl, hr = _tc_bn(z2, st, bn_g[l], bn_b[l], relu=(l < 2))

    p3l, p3r = _seg_sum_sc(hl, hr, srca, dsta, zeros_l)
    y3l, y3r, r3 = _tc_conv3(p3l, p3r, c30, c31, iso_type_3,
                             W_rel3[:_D], W_rel3[_D:],
                             W_root3[:_D], W_root3[_D:], b_rel3)
    a3l, a3r = _seg_sum_sc(y3l, y3r, src3, dst3, zeros_l)
    y4l, y4r, r4 = _tc_conv4(a3l, a3r, r3, W_rel4, W_root4, b_rel4)
    a4l, a4r = _seg_sum_sc(y4l, y4r, src3, dst3, zeros_l)

    out = _tc_readout(hl, hr, a4l, a4r, r4,
                      batch.reshape(_G, 1, _R), batch_3.reshape(_G, 1, _R),
                      W_out0[:_D], W_out0[_D:], b_out0,
                      W_out1, b_out1, W_out2, b_out2, W_out3, b_out3)
    return out.reshape(-1)
